# baseline (speedup 1.0000x reference)
import functools

import jax
import jax.numpy as jnp
from jax import lax
from jax.experimental import pallas as pl
from jax.experimental.pallas import tpu as pltpu
from jax.experimental.pallas import tpu_sc as plsc

_NC, _NS, _L = 2, 16, 16
_NW = _NC * _NS
_SIGN = 0x80000000


def _search16(tab_ref, rep_off, qkey, qf, base, n, steps, idx_bias):
    lo = jnp.full((_L,), base, jnp.int32)
    hi = jnp.full((_L,), base + n, jnp.int32)
    for _ in range(steps):
        mid = jnp.minimum((lo + hi) >> 1, base + n - 1)
        v = plsc.bitcast(plsc.load_gather(tab_ref, [mid]), jnp.uint32)
        pred = v < qkey
        lo = jnp.where(pred, mid + 1, lo)
        hi = jnp.where(pred, hi, mid)
    k = lo
    left = jnp.maximum(k - 1, base)
    right = jnp.minimum(k, base + n - 1)
    mag = jnp.uint32(0x7FFFFFFF)
    vl = plsc.bitcast(
        plsc.bitcast(plsc.load_gather(tab_ref, [left]), jnp.uint32) & mag,
        jnp.float32)
    vr = plsc.bitcast(
        plsc.bitcast(plsc.load_gather(tab_ref, [right]), jnp.uint32) & mag,
        jnp.float32)
    inf = jnp.full((_L,), jnp.inf, jnp.float32)
    dl = jnp.where(k == base, inf, jnp.abs(vl - qf))
    dr = jnp.where(k == base + n, inf, jnp.abs(vr - qf))
    rl = plsc.load_gather(tab_ref, [left + rep_off]) & 0xFFF
    rr = plsc.load_gather(tab_ref, [right + rep_off]) & 0xFFF
    idx = jnp.where(dl < dr, rl,
                    jnp.where(dr < dl, rr, jnp.minimum(rl, rr)))
    return idx - idx_bias


@functools.cache
def _build_sc_kernel(q, w, h):
    qw = q // _NW
    assert qw % 128 == 0
    n2 = w + h
    steps_w = w.bit_length()
    steps_h = h.bit_length()
    mesh = plsc.VectorSubcoreMesh(
        core_axis_name="c", subcore_axis_name="s",
        num_cores=_NC, num_subcores=_NS)

    @functools.partial(
        pl.kernel,
        out_type=jax.ShapeDtypeStruct((q,), jnp.float32),
        mesh=mesh,
        compiler_params=pltpu.CompilerParams(needs_layout_passes=False),
        scratch_types=[
            pltpu.VMEM((2 * n2,), jnp.int32),
            pltpu.VMEM((qw,), jnp.float32),
            pltpu.VMEM((qw,), jnp.float32),
            pltpu.VMEM((qw,), jnp.int32),
            pltpu.VMEM((qw,), jnp.float32),
            pltpu.SemaphoreType.DMA,
        ],
    )
    def sc_kernel(ll_hbm, tab_hbm, vw_hbm, out_hbm,
                  tab_v, lon_v, lat_v, idx_v, val_v, sem):
        wid = lax.axis_index("s") * _NC + lax.axis_index("c")
        base = wid * qw
        stage = [
            pltpu.async_copy(tab_hbm, tab_v, sem),
            pltpu.async_copy(ll_hbm.at[pl.ds(base, qw)], lon_v, sem),
            pltpu.async_copy(ll_hbm.at[pl.ds(q + base, qw)], lat_v, sem),
        ]
        for c in stage:
            c.wait()

        @plsc.parallel_loop(0, qw // _L, step=1, unroll=1)
        def q_body(t):
            off = t * _L
            lon16 = lon_v[pl.ds(off, _L)]
            lat16 = lat_v[pl.ds(off, _L)]
            klon = plsc.bitcast(lon16, jnp.uint32)
            klat = plsc.bitcast(lat16, jnp.uint32) | jnp.uint32(_SIGN)
            i_idx = _search16(tab_v, n2, klon, lon16, 0, w, steps_w, 0)
            j_idx = _search16(tab_v, n2, klat, lat16, w, h, steps_h, w)
            p = (((i_idx >> 3) * (h // 128) + (j_idx >> 7)) << 10) \
                | ((i_idx & 7) << 7) | (j_idx & 127)
            idx_v[pl.ds(off, _L)] = p

        gathers = [
            pltpu.async_copy(
                vw_hbm.at[idx_v.at[pl.ds(c * 128, 128)]],
                val_v.at[pl.ds(c * 128, 128)], sem)
            for c in range(qw // 128)
        ]
        for c in gathers:
            c.wait()
        pltpu.sync_copy(val_v, out_hbm.at[pl.ds(base, qw)])

    return sc_kernel


def kernel(lon, lat, x_coords, y_coords, vertex_weights):
    w, h = vertex_weights.shape
    q = lon.shape[0]
    n2 = w + h
    keys = jnp.concatenate([
        lax.bitcast_convert_type(x_coords, jnp.uint32),
        lax.bitcast_convert_type(y_coords, jnp.uint32) | jnp.uint32(_SIGN),
    ])
    iota = jnp.arange(n2, dtype=jnp.int32)
    svals, sidx = lax.sort((keys, iota), num_keys=2)
    new_run = jnp.concatenate(
        [jnp.ones((1,), jnp.bool_), svals[1:] != svals[:-1]])
    packed = jnp.where(new_run, (iota << 12) | sidx, 0)
    rep = lax.cummax(packed)
    keys_i32 = lax.bitcast_convert_type(svals, jnp.int32)
    vw_phys = vertex_weights.reshape(w // 8, 8, h // 128, 128) \
        .transpose(0, 2, 1, 3).reshape(-1)
    ll = jnp.concatenate([lon, lat])
    tab = jnp.concatenate([keys_i32, rep])
    sc = _build_sc_kernel(q, w, h)
    return sc(ll, tab, vw_phys)


# speedup vs baseline: 1.0387x; 1.0387x over previous
import functools

import jax
import jax.numpy as jnp
from jax import lax
from jax.experimental import pallas as pl
from jax.experimental.pallas import tpu as pltpu
from jax.experimental.pallas import tpu_sc as plsc

_NC, _NS, _L = 2, 16, 16
_NW = _NC * _NS
_SIGN = 0x80000000


def _search16(vals_ref, rep_ref, qkey, qf, base, n, steps, idx_bias):
    lo = jnp.full((_L,), base, jnp.int32)
    hi = jnp.full((_L,), base + n, jnp.int32)
    for _ in range(steps):
        mid = jnp.minimum((lo + hi) >> 1, base + n - 1)
        v = plsc.bitcast(plsc.load_gather(vals_ref, [mid]), jnp.uint32)
        pred = v < qkey
        lo = jnp.where(pred, mid + 1, lo)
        hi = jnp.where(pred, hi, mid)
    k = lo
    left = jnp.maximum(k - 1, base)
    right = jnp.minimum(k, base + n - 1)
    mag = jnp.uint32(0x7FFFFFFF)
    vl = plsc.bitcast(
        plsc.bitcast(plsc.load_gather(vals_ref, [left]), jnp.uint32) & mag,
        jnp.float32)
    vr = plsc.bitcast(
        plsc.bitcast(plsc.load_gather(vals_ref, [right]), jnp.uint32) & mag,
        jnp.float32)
    inf = jnp.full((_L,), jnp.inf, jnp.float32)
    dl = jnp.where(k == base, inf, jnp.abs(vl - qf))
    dr = jnp.where(k == base + n, inf, jnp.abs(vr - qf))
    rl = plsc.load_gather(rep_ref, [left]) & 0xFFF
    rr = plsc.load_gather(rep_ref, [right]) & 0xFFF
    idx = jnp.where(dl < dr, rl,
                    jnp.where(dr < dl, rr, jnp.minimum(rl, rr)))
    return idx - idx_bias


@functools.cache
def _build_sc_kernel(q, w, h):
    qw = q // _NW
    assert qw % 128 == 0
    n2 = w + h
    steps_w = w.bit_length()
    steps_h = h.bit_length()
    mesh = plsc.VectorSubcoreMesh(
        core_axis_name="c", subcore_axis_name="s",
        num_cores=_NC, num_subcores=_NS)

    @functools.partial(
        pl.kernel,
        out_type=jax.ShapeDtypeStruct((q,), jnp.float32),
        mesh=mesh,
        compiler_params=pltpu.CompilerParams(needs_layout_passes=False),
        scratch_types=[
            pltpu.VMEM((n2,), jnp.int32),
            pltpu.VMEM((n2,), jnp.int32),
            pltpu.VMEM((qw,), jnp.float32),
            pltpu.VMEM((qw,), jnp.float32),
            pltpu.VMEM((qw,), jnp.int32),
            pltpu.VMEM((qw,), jnp.float32),
            pltpu.SemaphoreType.DMA,
        ],
    )
    def sc_kernel(lon_hbm, lat_hbm, keys_hbm, rep_hbm, vw_hbm, out_hbm,
                  keys_v, rep_v, lon_v, lat_v, idx_v, val_v, sem):
        wid = lax.axis_index("s") * _NC + lax.axis_index("c")
        base = wid * qw
        stage = [
            pltpu.async_copy(keys_hbm, keys_v, sem),
            pltpu.async_copy(rep_hbm, rep_v, sem),
            pltpu.async_copy(lon_hbm.at[pl.ds(base, qw)], lon_v, sem),
            pltpu.async_copy(lat_hbm.at[pl.ds(base, qw)], lat_v, sem),
        ]
        for c in stage:
            c.wait()

        @plsc.parallel_loop(0, qw // _L, step=1, unroll=1)
        def q_body(t):
            off = t * _L
            lon16 = lon_v[pl.ds(off, _L)]
            lat16 = lat_v[pl.ds(off, _L)]
            klon = plsc.bitcast(lon16, jnp.uint32)
            klat = plsc.bitcast(lat16, jnp.uint32) | jnp.uint32(_SIGN)
            i_idx = _search16(keys_v, rep_v, klon, lon16, 0, w, steps_w, 0)
            j_idx = _search16(keys_v, rep_v, klat, lat16, w, h, steps_h, w)
            p = (((i_idx >> 3) * (h // 128) + (j_idx >> 7)) << 10) \
                | ((i_idx & 7) << 7) | (j_idx & 127)
            idx_v[pl.ds(off, _L)] = p

        gathers = [
            pltpu.async_copy(
                vw_hbm.at[idx_v.at[pl.ds(c * 128, 128)]],
                val_v.at[pl.ds(c * 128, 128)], sem)
            for c in range(qw // 128)
        ]
        for c in gathers:
            c.wait()
        pltpu.sync_copy(val_v, out_hbm.at[pl.ds(base, qw)])

    return sc_kernel


def kernel(lon, lat, x_coords, y_coords, vertex_weights):
    w, h = vertex_weights.shape
    q = lon.shape[0]
    n2 = w + h
    keys = jnp.concatenate([
        lax.bitcast_convert_type(x_coords, jnp.uint32),
        lax.bitcast_convert_type(y_coords, jnp.uint32) | jnp.uint32(_SIGN),
    ])
    iota = jnp.arange(n2, dtype=jnp.int32)
    svals, sidx = lax.sort((keys, iota), num_keys=2)
    new_run = jnp.concatenate(
        [jnp.ones((1,), jnp.bool_), svals[1:] != svals[:-1]])
    packed = jnp.where(new_run, (iota << 12) | sidx, 0)
    rep = lax.cummax(packed)
    keys_i32 = lax.bitcast_convert_type(svals, jnp.int32)
    vw_phys = vertex_weights.reshape(w // 8, 8, h // 128, 128) \
        .transpose(0, 2, 1, 3).reshape(-1)
    sc = _build_sc_kernel(q, w, h)
    return sc(lon, lat, keys_i32, rep, vw_phys)
